# baseline (speedup 1.0000x reference)
import jax
import jax.numpy as jnp
from jax.experimental import pallas as pl

TOKEN_BLOCK = 1024


def _moe_kernel(x_ref, gate_w_ref, gate_b_ref, ew_ref, eb_ref, out_ref):
    x = x_ref[...]
    logits = jnp.dot(x, gate_w_ref[...], preferred_element_type=jnp.float32)
    logits = logits + gate_b_ref[...]
    m = jnp.max(logits, axis=-1, keepdims=True)
    ex = jnp.exp(logits - m)
    w = ex / jnp.sum(ex, axis=-1, keepdims=True)

    acc = jnp.dot(w, eb_ref[...], preferred_element_type=jnp.float32)

    num_expert = eb_ref.shape[0]
    f_in = x.shape[1]
    for e in range(num_expert):
        pe = jnp.dot(x, ew_ref[e * f_in:(e + 1) * f_in, :],
                     preferred_element_type=jnp.float32)
        acc = acc + w[:, e:e + 1] * pe
    out_ref[...] = acc


def kernel(X, gate_W, gate_b, expert_W, expert_b):
    tokens, f_in = X.shape
    num_expert, _, f_out = expert_W.shape
    x_bf = X.astype(jnp.bfloat16)
    gw_bf = gate_W.astype(jnp.bfloat16)
    ew_bf = expert_W.astype(jnp.bfloat16).reshape(num_expert * f_in, f_out)
    gate_b2 = gate_b.reshape(1, num_expert)

    grid = (tokens // TOKEN_BLOCK,)
    return pl.pallas_call(
        _moe_kernel,
        grid=grid,
        in_specs=[
            pl.BlockSpec((TOKEN_BLOCK, f_in), lambda i: (i, 0)),
            pl.BlockSpec((f_in, num_expert), lambda i: (0, 0)),
            pl.BlockSpec((1, num_expert), lambda i: (0, 0)),
            pl.BlockSpec((num_expert * f_in, f_out), lambda i: (0, 0)),
            pl.BlockSpec((num_expert, f_out), lambda i: (0, 0)),
        ],
        out_specs=pl.BlockSpec((TOKEN_BLOCK, f_out), lambda i: (i, 0)),
        out_shape=jax.ShapeDtypeStruct((tokens, f_out), jnp.float32),
    )(x_bf, gw_bf, gate_b2, ew_bf, expert_b)


# speedup vs baseline: 1.0761x; 1.0761x over previous
import jax
import jax.numpy as jnp
from jax.experimental import pallas as pl

TOKEN_BLOCK = 1024


def _moe_kernel(x_ref, gate_w_ref, gate_b_ref, ew_ref, eb_ref, out_ref):
    x = x_ref[...]
    logits = jnp.dot(x, gate_w_ref[...], preferred_element_type=jnp.float32)
    logits = logits + gate_b_ref[...]
    m = jnp.max(logits, axis=-1, keepdims=True)
    ex = jnp.exp(logits - m)
    w = ex / jnp.sum(ex, axis=-1, keepdims=True)

    acc = jnp.dot(w, eb_ref[...], preferred_element_type=jnp.float32)

    x_bf = x.astype(jnp.bfloat16)
    num_expert = eb_ref.shape[0]
    f_in = x.shape[1]
    for e in range(num_expert):
        pe = jnp.dot(x_bf, ew_ref[e * f_in:(e + 1) * f_in, :],
                     preferred_element_type=jnp.float32)
        acc = acc + w[:, e:e + 1] * pe
    out_ref[...] = acc


def kernel(X, gate_W, gate_b, expert_W, expert_b):
    tokens, f_in = X.shape
    num_expert, _, f_out = expert_W.shape
    ew_bf = expert_W.astype(jnp.bfloat16).reshape(num_expert * f_in, f_out)
    gate_b2 = gate_b.reshape(1, num_expert)

    grid = (tokens // TOKEN_BLOCK,)
    return pl.pallas_call(
        _moe_kernel,
        grid=grid,
        in_specs=[
            pl.BlockSpec((TOKEN_BLOCK, f_in), lambda i: (i, 0)),
            pl.BlockSpec((f_in, num_expert), lambda i: (0, 0)),
            pl.BlockSpec((1, num_expert), lambda i: (0, 0)),
            pl.BlockSpec((num_expert * f_in, f_out), lambda i: (0, 0)),
            pl.BlockSpec((num_expert, f_out), lambda i: (0, 0)),
        ],
        out_specs=pl.BlockSpec((TOKEN_BLOCK, f_out), lambda i: (i, 0)),
        out_shape=jax.ShapeDtypeStruct((tokens, f_out), jnp.float32),
    )(X, gate_W, gate_b2, ew_bf, expert_b)


# speedup vs baseline: 1.1023x; 1.0244x over previous
import jax
import jax.numpy as jnp
from jax.experimental import pallas as pl
from jax.experimental.pallas import tpu as pltpu

TOKEN_BLOCK = 1024


def _moe_kernel(x_ref, gate_w_ref, gate_b_ref, ew_hbm_ref, eb_ref, out_ref,
                wbf_ref, land_ref, sem):
    num_expert = eb_ref.shape[0]
    f_in = x_ref.shape[1]

    @pl.when(pl.program_id(0) == 0)
    def _load_and_cast_weights():
        def copy(e):
            return pltpu.make_async_copy(
                ew_hbm_ref.at[e], land_ref.at[e % 2], sem.at[e % 2])
        copy(0).start()
        for e in range(num_expert):
            copy(e).wait()
            if e + 1 < num_expert:
                copy(e + 1).start()
            wbf_ref[e * f_in:(e + 1) * f_in, :] = (
                land_ref[e % 2].astype(jnp.bfloat16))

    x = x_ref[...]
    logits = jnp.dot(x, gate_w_ref[...], preferred_element_type=jnp.float32)
    logits = logits + gate_b_ref[...]
    m = jnp.max(logits, axis=-1, keepdims=True)
    ex = jnp.exp(logits - m)
    w = ex / jnp.sum(ex, axis=-1, keepdims=True)

    acc = jnp.dot(w, eb_ref[...], preferred_element_type=jnp.float32)

    x_bf = x.astype(jnp.bfloat16)
    for e in range(num_expert):
        pe = jnp.dot(x_bf, wbf_ref[e * f_in:(e + 1) * f_in, :],
                     preferred_element_type=jnp.float32)
        acc = acc + w[:, e:e + 1] * pe
    out_ref[...] = acc


def kernel(X, gate_W, gate_b, expert_W, expert_b):
    tokens, f_in = X.shape
    num_expert, _, f_out = expert_W.shape
    gate_b2 = gate_b.reshape(1, num_expert)

    grid = (tokens // TOKEN_BLOCK,)
    return pl.pallas_call(
        _moe_kernel,
        grid=grid,
        in_specs=[
            pl.BlockSpec((TOKEN_BLOCK, f_in), lambda i: (i, 0)),
            pl.BlockSpec((f_in, num_expert), lambda i: (0, 0)),
            pl.BlockSpec((1, num_expert), lambda i: (0, 0)),
            pl.BlockSpec(memory_space=pl.ANY),
            pl.BlockSpec((num_expert, f_out), lambda i: (0, 0)),
        ],
        out_specs=pl.BlockSpec((TOKEN_BLOCK, f_out), lambda i: (i, 0)),
        out_shape=jax.ShapeDtypeStruct((tokens, f_out), jnp.float32),
        scratch_shapes=[
            pltpu.VMEM((num_expert * f_in, f_out), jnp.bfloat16),
            pltpu.VMEM((2, f_in, f_out), jnp.float32),
            pltpu.SemaphoreType.DMA((2,)),
        ],
    )(X, gate_W, gate_b2, expert_W, expert_b)


# speedup vs baseline: 1.1053x; 1.0027x over previous
import jax
import jax.numpy as jnp
from jax.experimental import pallas as pl
from jax.experimental.pallas import tpu as pltpu

TOKEN_BLOCK = 1024


def _moe_kernel(x_ref, gate_w_ref, gate_b_ref, ew_hbm_ref, eb_ref, out_ref,
                wbf_ref, land_ref, sem):
    num_expert = eb_ref.shape[0]
    f_in = x_ref.shape[1]

    @pl.when(pl.program_id(0) == 0)
    def _load_and_cast_weights():
        def copy(e):
            return pltpu.make_async_copy(
                ew_hbm_ref.at[e], land_ref.at[e % 2], sem.at[e % 2])
        copy(0).start()
        for e in range(num_expert):
            copy(e).wait()
            if e + 1 < num_expert:
                copy(e + 1).start()
            wbf_ref[e * f_in:(e + 1) * f_in, :] = (
                land_ref[e % 2].astype(jnp.bfloat16))

    x = x_ref[...]
    x_bf = x.astype(jnp.bfloat16)
    gw_bf = gate_w_ref[...].astype(jnp.bfloat16)
    logits = jnp.dot(x_bf, gw_bf, preferred_element_type=jnp.float32)
    logits = logits + gate_b_ref[...]
    m = jnp.max(logits, axis=-1, keepdims=True)
    ex = jnp.exp(logits - m)
    w = ex / jnp.sum(ex, axis=-1, keepdims=True)

    acc0 = jnp.dot(w, eb_ref[...], preferred_element_type=jnp.float32)
    acc1 = jnp.zeros_like(acc0)

    accs = [acc0, acc1]
    for e in range(num_expert):
        pe = jnp.dot(x_bf, wbf_ref[e * f_in:(e + 1) * f_in, :],
                     preferred_element_type=jnp.float32)
        accs[e % 2] = accs[e % 2] + w[:, e:e + 1] * pe
    out_ref[...] = accs[0] + accs[1]


def kernel(X, gate_W, gate_b, expert_W, expert_b):
    tokens, f_in = X.shape
    num_expert, _, f_out = expert_W.shape
    gate_b2 = gate_b.reshape(1, num_expert)

    grid = (tokens // TOKEN_BLOCK,)
    return pl.pallas_call(
        _moe_kernel,
        grid=grid,
        in_specs=[
            pl.BlockSpec((TOKEN_BLOCK, f_in), lambda i: (i, 0)),
            pl.BlockSpec((f_in, num_expert), lambda i: (0, 0)),
            pl.BlockSpec((1, num_expert), lambda i: (0, 0)),
            pl.BlockSpec(memory_space=pl.ANY),
            pl.BlockSpec((num_expert, f_out), lambda i: (0, 0)),
        ],
        out_specs=pl.BlockSpec((TOKEN_BLOCK, f_out), lambda i: (i, 0)),
        out_shape=jax.ShapeDtypeStruct((tokens, f_out), jnp.float32),
        scratch_shapes=[
            pltpu.VMEM((num_expert * f_in, f_out), jnp.bfloat16),
            pltpu.VMEM((2, f_in, f_out), jnp.float32),
            pltpu.SemaphoreType.DMA((2,)),
        ],
    )(X, gate_W, gate_b2, expert_W, expert_b)
